# baseline (speedup 1.0000x reference)
import functools

import jax
import jax.numpy as jnp
from jax import lax
from jax.experimental import pallas as pl
from jax.experimental.pallas import tpu as pltpu
from jax.experimental.pallas import tpu_sc as plsc

NC = 2
NS = 16
NW = NC * NS
LANES = 128


@functools.lru_cache(maxsize=None)
def _make_gather2(b, s, d):
    assert b == NW * LANES
    assert d % 16 == 0
    eg = d // 8
    gp = d // 16
    mesh = plsc.VectorSubcoreMesh(core_axis_name="c", subcore_axis_name="s")
    plane = jax.ShapeDtypeStruct((s, eg, NW, 8 * LANES), jnp.float32)

    @functools.partial(
        pl.kernel,
        mesh=mesh,
        out_type=[plane, plane],
        scratch_types=[
            pltpu.VMEM((s, LANES), jnp.int32),
            pltpu.VMEM((2, LANES, d), jnp.float32),
            pltpu.VMEM((2, LANES, d), jnp.float32),
            pltpu.VMEM((2, LANES, d), jnp.float32),
            pltpu.VMEM((2, LANES, d), jnp.float32),
            pltpu.VMEM((2, d * LANES), jnp.float32),
            pltpu.VMEM((2, d * LANES), jnp.float32),
            pltpu.SemaphoreType.DMA,
            pltpu.SemaphoreType.DMA,
            pltpu.SemaphoreType.DMA,
            pltpu.SemaphoreType.DMA,
            pltpu.SemaphoreType.DMA,
        ],
        compiler_params=pltpu.CompilerParams(use_tc_tiling_on_sc=False,
                                             needs_layout_passes=False),
    )
    def gather2(xs_hbm, wr_hbm, wi_hbm, pr_hbm, pi_hbm,
                idx_v, br0, br1, bi0, bi1, str_, sti,
                gr0, gr1, gi0, gi1, oss):
        w = lax.axis_index("s") * NC + lax.axis_index("c")
        brs = (br0, br1)
        bis = (bi0, bi1)
        grs = (gr0, gr1)
        gis = (gi0, gi1)
        iota16 = lax.iota(jnp.int32, 16)
        posbase = [(16 * k + iota16) * LANES for k in range(gp)]

        pltpu.sync_copy(xs_hbm.at[:, pl.ds(w * LANES, LANES)], idx_v)

        def start_gather(t, p):
            for h in range(2):
                idx1 = idx_v.at[2 * t + h]
                pltpu.async_copy(wr_hbm.at[idx1], brs[p].at[h], grs[p])
                pltpu.async_copy(wi_hbm.at[idx1], bis[p].at[h], gis[p])

        def wait_gather(t, p):
            for h in range(2):
                idx1 = idx_v.at[2 * t + h]
                pltpu.make_async_copy(wr_hbm.at[idx1], brs[p].at[h],
                                      grs[p]).wait()
                pltpu.make_async_copy(wi_hbm.at[idx1], bis[p].at[h],
                                      gis[p]).wait()

        def transpose(p, h):
            def row_body(l, carry):
                for k in range(gp):
                    pos = posbase[k] + l
                    vr = brs[p][h, l, pl.ds(16 * k, 16)]
                    plsc.store_scatter(str_.at[h], [pos], vr)
                    vi = bis[p][h, l, pl.ds(16 * k, 16)]
                    plsc.store_scatter(sti.at[h], [pos], vi)
                return carry

            lax.fori_loop(0, LANES, row_body, 0, unroll=False)

        def fire_out(si, h):
            def g_body(g, carry):
                pltpu.async_copy(str_.at[h, pl.ds(g * 8 * LANES, 8 * LANES)],
                                 pr_hbm.at[si, g, w], oss)
                pltpu.async_copy(sti.at[h, pl.ds(g * 8 * LANES, 8 * LANES)],
                                 pi_hbm.at[si, g, w], oss)
                return carry

            lax.fori_loop(0, eg, g_body, 0, unroll=False)

        def drain_out():
            def g_body(g, carry):
                for h in range(2):
                    pltpu.make_async_copy(str_.at[h, pl.ds(0, 8 * LANES)],
                                          pr_hbm.at[0, 0, 0], oss).wait()
                    pltpu.make_async_copy(sti.at[h, pl.ds(0, 8 * LANES)],
                                          pi_hbm.at[0, 0, 0], oss).wait()
                return carry

            lax.fori_loop(0, eg, g_body, 0, unroll=False)

        def step(t, p, start_next, drain_first):
            wait_gather(t, p)
            if start_next is not None:
                start_gather(start_next, 1 - p)
            if drain_first:
                drain_out()
            transpose(p, 0)
            transpose(p, 1)
            fire_out(2 * t, 0)
            fire_out(2 * t + 1, 1)

        nstep = s // 2
        assert s % 2 == 0 and nstep % 2 == 1 and nstep >= 5
        start_gather(0, 0)
        step(0, 0, 1, False)

        def pair_body(t2, carry):
            step(2 * t2 + 1, 1, 2 * t2 + 2, True)
            step(2 * t2 + 2, 0, 2 * t2 + 3, True)
            return carry

        lax.fori_loop(0, (nstep - 3) // 2, pair_body, 0, unroll=False)

        step(nstep - 2, 1, nstep - 1, True)
        step(nstep - 1, 0, None, True)

        drain_out()

    return gather2


def kernel(x, W_real, W_imag):
    b, s = x.shape
    d = W_real.shape[1]
    xs = jnp.swapaxes(x, 0, 1).astype(jnp.int32)
    pr, pi = _make_gather2(b, s, d)(xs, W_real, W_imag)
    r5 = pr.reshape(s, d // 8, NW, 8, LANES).transpose(2, 4, 0, 1, 3)
    i5 = pi.reshape(s, d // 8, NW, 8, LANES).transpose(2, 4, 0, 1, 3)
    return lax.complex(r5.reshape(b, s, d), i5.reshape(b, s, d))


# speedup vs baseline: 1.0118x; 1.0118x over previous
import functools

import jax
import jax.numpy as jnp
from jax import lax
from jax.experimental import pallas as pl
from jax.experimental.pallas import tpu as pltpu
from jax.experimental.pallas import tpu_sc as plsc

NC = 2
NS = 16
NW = NC * NS
LANES = 128


@functools.lru_cache(maxsize=None)
def _make_gather2(b, s, d):
    assert b == NW * LANES
    assert d % 16 == 0
    eg = d // 8
    gp = d // 16
    mesh = plsc.VectorSubcoreMesh(core_axis_name="c", subcore_axis_name="s")
    plane = jax.ShapeDtypeStruct((s, eg, NW, 8 * LANES), jnp.float32)

    @functools.partial(
        pl.kernel,
        mesh=mesh,
        out_type=[plane, plane],
        scratch_types=[
            pltpu.VMEM((s, LANES), jnp.int32),
            pltpu.VMEM((LANES, d), jnp.float32),
            pltpu.VMEM((LANES, d), jnp.float32),
            pltpu.VMEM((LANES, d), jnp.float32),
            pltpu.VMEM((LANES, d), jnp.float32),
            pltpu.VMEM((d * LANES,), jnp.float32),
            pltpu.VMEM((d * LANES,), jnp.float32),
            pltpu.VMEM((d * LANES,), jnp.float32),
            pltpu.VMEM((d * LANES,), jnp.float32),
            pltpu.SemaphoreType.DMA,
            pltpu.SemaphoreType.DMA,
            pltpu.SemaphoreType.DMA,
            pltpu.SemaphoreType.DMA,
            pltpu.SemaphoreType.DMA,
            pltpu.SemaphoreType.DMA,
        ],
        compiler_params=pltpu.CompilerParams(use_tc_tiling_on_sc=False,
                                             needs_layout_passes=False),
    )
    def gather2(xs_hbm, wr_hbm, wi_hbm, pr_hbm, pi_hbm,
                idx_v, br0, br1, bi0, bi1, sr0, sr1, si0, si1,
                gr0, gr1, gi0, gi1, os0, os1):
        w = lax.axis_index("s") * NC + lax.axis_index("c")
        brs = (br0, br1)
        bis = (bi0, bi1)
        srs = (sr0, sr1)
        sis = (si0, si1)
        grs = (gr0, gr1)
        gis = (gi0, gi1)
        oss = (os0, os1)
        iota16 = lax.iota(jnp.int32, 16)
        posbase = [(16 * k + iota16) * LANES for k in range(gp)]

        pltpu.sync_copy(xs_hbm.at[:, pl.ds(w * LANES, LANES)], idx_v)

        def start_gather(si, p):
            pltpu.async_copy(wr_hbm.at[idx_v.at[si]], brs[p], grs[p])
            pltpu.async_copy(wi_hbm.at[idx_v.at[si]], bis[p], gis[p])

        def wait_gather(p):
            pltpu.make_async_copy(wr_hbm.at[pl.ds(0, LANES)], brs[p],
                                  grs[p]).wait()
            pltpu.make_async_copy(wi_hbm.at[pl.ds(0, LANES)], bis[p],
                                  gis[p]).wait()

        def transpose(p):
            def row_body(l, carry):
                for k in range(gp):
                    pos = posbase[k] + l
                    vr = brs[p][l, pl.ds(16 * k, 16)]
                    plsc.store_scatter(srs[p], [pos], vr)
                    vi = bis[p][l, pl.ds(16 * k, 16)]
                    plsc.store_scatter(sis[p], [pos], vi)
                return carry

            lax.fori_loop(0, LANES, row_body, 0, unroll=False)

        def fire_out(si, p):
            def g_body(g, carry):
                pltpu.async_copy(srs[p].at[pl.ds(g * 8 * LANES, 8 * LANES)],
                                 pr_hbm.at[si, g, w], oss[p])
                pltpu.async_copy(sis[p].at[pl.ds(g * 8 * LANES, 8 * LANES)],
                                 pi_hbm.at[si, g, w], oss[p])
                return carry

            lax.fori_loop(0, eg, g_body, 0, unroll=False)

        def drain_out(p):
            def g_body(g, carry):
                pltpu.make_async_copy(srs[p].at[pl.ds(0, 8 * LANES)],
                                      pr_hbm.at[0, 0, 0], oss[p]).wait()
                pltpu.make_async_copy(sis[p].at[pl.ds(0, 8 * LANES)],
                                      pi_hbm.at[0, 0, 0], oss[p]).wait()
                return carry

            lax.fori_loop(0, eg, g_body, 0, unroll=False)

        def step(si, p, start_next, drain_first):
            q = 1 - p
            wait_gather(p)
            if start_next is not None:
                start_gather(start_next, q)
            if drain_first:
                drain_out(p)
            transpose(p)
            fire_out(si, p)

        assert s % 2 == 0 and s >= 6
        start_gather(0, 0)
        step(0, 0, 1, False)
        step(1, 1, 2, False)

        def pair_body(t, carry):
            si = 2 * t
            step(si, 0, si + 1, True)
            step(si + 1, 1, si + 2, True)
            return carry

        lax.fori_loop(1, s // 2 - 1, pair_body, 0, unroll=False)

        step(s - 2, 0, s - 1, True)
        step(s - 1, 1, None, True)

        drain_out(0)
        drain_out(1)

    return gather2


def kernel(x, W_real, W_imag):
    b, s = x.shape
    d = W_real.shape[1]
    xs = jnp.swapaxes(x, 0, 1).astype(jnp.int32)
    pr, pi = _make_gather2(b, s, d)(xs, W_real, W_imag)
    r5 = pr.reshape(s, d // 8, NW, 8, LANES).transpose(2, 4, 0, 1, 3)
    i5 = pi.reshape(s, d // 8, NW, 8, LANES).transpose(2, 4, 0, 1, 3)
    return lax.complex(r5.reshape(b, s, d), i5.reshape(b, s, d))
